# baseline (speedup 1.0000x reference)
import functools

import jax
import jax.numpy as jnp
from jax import lax
from jax.experimental import pallas as pl
from jax.experimental.pallas import tpu as pltpu
from jax.experimental.pallas import tpu_sc as plsc

VOCAB = 1000000
EMBED_DIM = 32
BATCH = 16384

NC = 2
NS = 16
L = 16
NW = NC * NS
B_PER_W = BATCH // NW
CHUNK = 128
N_CHUNKS = B_PER_W // CHUNK
GROUPS = B_PER_W // L


def _sc_body(emb_hbm, tgt_hbm, ctx_hbm, out_hbm,
             idx_t, idx_c, rows_t, rows_c, out_v, sem):
    wid = lax.axis_index("s") * NC + lax.axis_index("c")
    base = wid * B_PER_W

    pltpu.sync_copy(tgt_hbm.at[pl.ds(base, B_PER_W)], idx_t)
    pltpu.sync_copy(ctx_hbm.at[pl.ds(base, B_PER_W)], idx_c)

    copies = []
    for j in range(N_CHUNKS):
        sl = pl.ds(j * CHUNK, CHUNK)
        copies.append(pltpu.async_copy(emb_hbm.at[idx_t.at[sl]], rows_t.at[sl], sem))
        copies.append(pltpu.async_copy(emb_hbm.at[idx_c.at[sl]], rows_c.at[sl], sem))
    for c in copies:
        c.wait()

    lanes = jnp.arange(L, dtype=jnp.int32)

    def group_body(g, carry):
        acc = jnp.zeros((L,), jnp.float32)
        for r in range(L):
            row = g * L + r
            s = (rows_t[row, pl.ds(0, L)] * rows_c[row, pl.ds(0, L)]
                 + rows_t[row, pl.ds(L, L)] * rows_c[row, pl.ds(L, L)])
            acc = jnp.where(lanes == r, jnp.sum(s), acc)
        out_v[pl.ds(g * L, L)] = acc
        return carry

    lax.fori_loop(0, GROUPS, group_body, 0)

    pltpu.sync_copy(out_v, out_hbm.at[pl.ds(base, B_PER_W)])


@jax.jit
def _word2vec_score(target_word, context_word, embeddings):
    mesh = plsc.VectorSubcoreMesh(core_axis_name="c", subcore_axis_name="s")
    k = functools.partial(
        pl.kernel,
        mesh=mesh,
        compiler_params=pltpu.CompilerParams(needs_layout_passes=False,
                                             use_tc_tiling_on_sc=False),
        out_type=jax.ShapeDtypeStruct((BATCH,), jnp.float32),
        scratch_types=[
            pltpu.VMEM((B_PER_W,), jnp.int32),
            pltpu.VMEM((B_PER_W,), jnp.int32),
            pltpu.VMEM((B_PER_W, EMBED_DIM), jnp.float32),
            pltpu.VMEM((B_PER_W, EMBED_DIM), jnp.float32),
            pltpu.VMEM((B_PER_W,), jnp.float32),
            pltpu.SemaphoreType.DMA,
        ],
    )(_sc_body)
    return k(embeddings, target_word, context_word)


def kernel(target_word, context_word, embeddings):
    return _word2vec_score(target_word.astype(jnp.int32),
                           context_word.astype(jnp.int32),
                           embeddings)
